# baseline (speedup 1.0000x reference)
import functools

import jax
import jax.numpy as jnp
from jax import lax
from jax.experimental import pallas as pl
from jax.experimental.pallas import tpu as pltpu
from jax.experimental.pallas import tpu_sc as plsc

N = 10000
E = 320000
D = 128

NC = 2
NS = 16
NW = NC * NS
EPW = E // NW
K = 40
NBUF = 5
DH = D // 2
GB = 125
C = EPW // K
NP = 10240
RPT = NP // NS
RB = 128
DEGW = 16

_sc_mesh = plsc.VectorSubcoreMesh(core_axis_name="c", subcore_axis_name="s")


@functools.partial(
    pl.kernel,
    out_type=(jax.ShapeDtypeStruct((NP, DEGW), jnp.float32),
              jax.ShapeDtypeStruct((NP, DEGW), jnp.float32)),
    mesh=_sc_mesh,
    compiler_params=pltpu.CompilerParams(use_tc_tiling_on_sc=False),
    scratch_types=[
        pltpu.VMEM_SHARED((NP, DEGW), jnp.float32),
        pltpu.VMEM((C, K), jnp.int32),
        pltpu.VMEM((K, DEGW), jnp.float32),
        pltpu.VMEM((RB, DEGW), jnp.float32),
    ],
)
def _sc_degree(edges_hbm, ones_hbm, zeros_hbm, outA, outB, acc, idx_v, ones_v,
               stage):
    c = lax.axis_index("c")
    s = lax.axis_index("s")
    wid = s * NC + c
    pltpu.sync_copy(zeros_hbm, stage)
    for i in range(RPT // RB):
        pltpu.sync_copy(stage, acc.at[pl.ds(s * RPT + i * RB, RB)])
    pltpu.sync_copy(ones_hbm, ones_v)
    pltpu.sync_copy(edges_hbm.at[1, wid], idx_v)
    plsc.subcore_barrier()

    def body(j, _):
        pltpu.sync_copy(ones_v, acc.at[idx_v.at[j]], add=True)
        return 0

    lax.fori_loop(0, C, body, 0)
    plsc.subcore_barrier()
    for i in range(RPT // RB):
        sl = pl.ds(s * RPT + i * RB, RB)
        pltpu.sync_copy(acc.at[sl], stage)

        @pl.when(c == 0)
        def _():
            pltpu.sync_copy(stage, outA.at[sl])

        @pl.when(c == 1)
        def _():
            pltpu.sync_copy(stage, outB.at[sl])


@functools.partial(
    pl.kernel,
    out_type=(jax.ShapeDtypeStruct((NP, D), jnp.float32),
              jax.ShapeDtypeStruct((NP, D), jnp.float32)),
    mesh=_sc_mesh,
    compiler_params=pltpu.CompilerParams(use_tc_tiling_on_sc=False),
    scratch_types=[
        pltpu.VMEM_SHARED((NP, DH), jnp.float32),
        pltpu.VMEM_SHARED((N, DH), jnp.float32),
        pltpu.VMEM((C, K), jnp.int32),
        pltpu.VMEM((C, K), jnp.int32),
        pltpu.VMEM((GB, DH), jnp.float32),
    ]
    + [pltpu.VMEM((K, DH), jnp.float32) for _ in range(NBUF)]
    + [pltpu.SemaphoreType.DMA for _ in range(2 * NBUF)],
)
def _sc_scatter(g_hbm, edges_hbm, zeros_hbm, outA, outB,
                acc, gtab, src_v, dst_v, gbuf, *bufs_and_sems):
    rows = bufs_and_sems[:NBUF]
    gsem = bufs_and_sems[NBUF:2 * NBUF]
    ssem = bufs_and_sems[2 * NBUF:]
    c = lax.axis_index("c")
    s = lax.axis_index("s")
    wid = s * NC + c
    pltpu.sync_copy(edges_hbm.at[0, wid], src_v)
    pltpu.sync_copy(edges_hbm.at[1, wid], dst_v)

    for h in range(2):
        for i in range(N // NS // GB):
            r0 = s * (N // NS) + i * GB
            pltpu.sync_copy(g_hbm.at[pl.ds(r0, GB), pl.ds(h * DH, DH)], gbuf)
            pltpu.sync_copy(gbuf, gtab.at[pl.ds(r0, GB)])
        pltpu.sync_copy(zeros_hbm, rows[0])
        for i in range(RPT // K):
            pltpu.sync_copy(rows[0], acc.at[pl.ds(s * RPT + i * K, K)])
        for t in range(NBUF):
            pltpu.async_copy(gtab.at[src_v.at[t]], rows[t], gsem[t])
        plsc.subcore_barrier()

        def body(i, _):
            ds = []
            for t in range(NBUF):
                j = NBUF * i + t
                pltpu.make_async_copy(gtab.at[src_v.at[j]], rows[t],
                                      gsem[t]).wait()
                ds.append(pltpu.async_copy(rows[t], acc.at[dst_v.at[j]],
                                           ssem[t], add=True))
            for t in range(NBUF):
                ds[t].wait()

                @pl.when(i < C // NBUF - 1)
                def _():
                    j2 = NBUF * i + NBUF + t
                    pltpu.async_copy(gtab.at[src_v.at[j2]], rows[t], gsem[t])
            return 0

        lax.fori_loop(0, C // NBUF, body, 0)
        plsc.subcore_barrier()
        for i in range(RPT // K):
            sl = pl.ds(s * RPT + i * K, K)
            pltpu.sync_copy(acc.at[sl], rows[0])
            cols = pl.ds(h * DH, DH)

            @pl.when(c == 0)
            def _():
                pltpu.sync_copy(rows[0], outA.at[sl, cols])

            @pl.when(c == 1)
            def _():
                pltpu.sync_copy(rows[0], outB.at[sl, cols])
        plsc.subcore_barrier()


R = 1000
GRID = N // R


def _dis(degA, degB):
    return lax.rsqrt(degA[:, :1] + degB[:, :1] + 1.0)


def _tc_pre_body(x_ref, w_ref, degA_ref, degB_ref, g_ref):
    g = jnp.dot(x_ref[...], w_ref[...], preferred_element_type=jnp.float32)
    g_ref[...] = g * _dis(degA_ref[...], degB_ref[...])


def _tc_post_body(accA_ref, accB_ref, g_ref, degA_ref, degB_ref, b_ref,
                  out_ref, sums_ref):
    i = pl.program_id(0)
    dis = _dis(degA_ref[...], degB_ref[...])
    out = (accA_ref[...] + accB_ref[...] + g_ref[...]) * dis + b_ref[...]
    out_ref[...] = out

    @pl.when(i == 0)
    def _():
        sums_ref[...] = jnp.zeros_like(sums_ref)

    sums_ref[0:1, :] += jnp.sum(out, axis=0, keepdims=True)
    sums_ref[1:2, :] += jnp.sum(out * out, axis=0, keepdims=True)


def _graph_norm(x, sums, w, b, ms, eps=1e-5):
    mean = sums[0:1, :] * (1.0 / N)
    ex2 = sums[1:2, :] * (1.0 / N)
    var = ex2 - mean * mean * ms * (2.0 - ms)
    return w * (x - mean * ms) / jnp.sqrt(var + eps) + b


def _tc_gn_mm_body(x_ref, sums_ref, w2_ref, degA_ref, degB_ref,
                   gnw_ref, gnb_ref, gnms_ref, g2_ref):
    y = jnp.maximum(
        _graph_norm(x_ref[...], sums_ref[...], gnw_ref[...], gnb_ref[...],
                    gnms_ref[...]), 0.0)
    h = jnp.dot(y, w2_ref[...], preferred_element_type=jnp.float32)
    g2_ref[...] = h * _dis(degA_ref[...], degB_ref[...])


def _tc_final_body(x_ref, sums_ref, gnw_ref, gnb_ref, gnms_ref,
                   lw1_ref, lb1_ref, lw2_ref, lb2_ref, y_ref):
    y = jnp.maximum(
        _graph_norm(x_ref[...], sums_ref[...], gnw_ref[...], gnb_ref[...],
                    gnms_ref[...]), 0.0)
    r = jnp.maximum(
        jnp.dot(y, lw1_ref[...], preferred_element_type=jnp.float32)
        + lb1_ref[...], 0.0)
    y_ref[...] = (jnp.dot(r, lw2_ref[...], preferred_element_type=jnp.float32)
                  + lb2_ref[...])


def _rows(shape):
    return pl.BlockSpec(shape, lambda i: (i, 0))


def _full(shape):
    return pl.BlockSpec(shape, lambda i: (0, 0))


_tc_pre = pl.pallas_call(
    _tc_pre_body,
    grid=(GRID,),
    in_specs=[_rows((R, D)), _full((D, D)), _rows((R, DEGW)), _rows((R, DEGW))],
    out_specs=_rows((R, D)),
    out_shape=jax.ShapeDtypeStruct((N, D), jnp.float32),
)

_tc_post = pl.pallas_call(
    _tc_post_body,
    grid=(GRID,),
    in_specs=[_rows((R, D)), _rows((R, D)), _rows((R, D)),
              _rows((R, DEGW)), _rows((R, DEGW)), _full((1, D))],
    out_specs=(_rows((R, D)), _full((8, D))),
    out_shape=(jax.ShapeDtypeStruct((N, D), jnp.float32),
               jax.ShapeDtypeStruct((8, D), jnp.float32)),
)

_tc_gn_mm = pl.pallas_call(
    _tc_gn_mm_body,
    grid=(GRID,),
    in_specs=[_rows((R, D)), _full((8, D)), _full((D, D)),
              _rows((R, DEGW)), _rows((R, DEGW)),
              _full((1, D)), _full((1, D)), _full((1, D))],
    out_specs=_rows((R, D)),
    out_shape=jax.ShapeDtypeStruct((N, D), jnp.float32),
)

_tc_final = pl.pallas_call(
    _tc_final_body,
    grid=(GRID,),
    in_specs=[_rows((R, D)), _full((8, D)),
              _full((1, D)), _full((1, D)), _full((1, D)),
              _full((D, D)), _full((1, D)), _full((D, D)), _full((1, D))],
    out_specs=_rows((R, D)),
    out_shape=jax.ShapeDtypeStruct((N, D), jnp.float32),
)


def kernel(x, edge_index, W1, b1, gn1_w, gn1_b, gn1_ms, W2, b2, gn2_w, gn2_b,
           gn2_ms, lW1, lb1, lW2, lb2):
    edges = edge_index.reshape(2, NW, C, K)
    ones16 = jnp.ones((K, DEGW), jnp.float32)
    zeros16 = jnp.zeros((RB, DEGW), jnp.float32)
    zerosD = jnp.zeros((K, DH), jnp.float32)
    row = lambda v: v.reshape(1, D)

    degA, degB = _sc_degree(edges, ones16, zeros16)

    g1 = _tc_pre(x, W1, degA, degB)
    a1, b1acc = _sc_scatter(g1, edges, zerosD)
    out1, sums1 = _tc_post(a1, b1acc, g1, degA, degB, row(b1))

    g2 = _tc_gn_mm(out1, sums1, W2, degA, degB,
                   row(gn1_w), row(gn1_b), row(gn1_ms))
    a2, b2acc = _sc_scatter(g2, edges, zerosD)
    out2, sums2 = _tc_post(a2, b2acc, g2, degA, degB, row(b2))

    return _tc_final(out2, sums2, row(gn2_w), row(gn2_b), row(gn2_ms),
                     lW1, row(lb1), lW2, row(lb2))


# speedup vs baseline: 1.4957x; 1.4957x over previous
import functools

import jax
import jax.numpy as jnp
from jax import lax
from jax.experimental import pallas as pl
from jax.experimental.pallas import tpu as pltpu
from jax.experimental.pallas import tpu_sc as plsc

N = 10000
E = 320000
D = 128

NC = 2
NS = 16
NW = NC * NS
EPW = E // NW
K = 40
NBUF = 5
C = EPW // K
NP = 10240
RPT = NP // NS
RB = 128
DEGW = 16

_sc_mesh = plsc.VectorSubcoreMesh(core_axis_name="c", subcore_axis_name="s")


@functools.partial(
    pl.kernel,
    out_type=(jax.ShapeDtypeStruct((NP, DEGW), jnp.float32),
              jax.ShapeDtypeStruct((NP, DEGW), jnp.float32)),
    mesh=_sc_mesh,
    compiler_params=pltpu.CompilerParams(use_tc_tiling_on_sc=False),
    scratch_types=[
        pltpu.VMEM_SHARED((NP, DEGW), jnp.float32),
        pltpu.VMEM((C, K), jnp.int32),
        pltpu.VMEM((K, DEGW), jnp.float32),
        pltpu.VMEM((RB, DEGW), jnp.float32),
    ],
)
def _sc_degree(edges_hbm, ones_hbm, zeros_hbm, outA, outB, acc, idx_v, ones_v,
               stage):
    c = lax.axis_index("c")
    s = lax.axis_index("s")
    wid = s * NC + c
    pltpu.sync_copy(zeros_hbm, stage)
    for i in range(RPT // RB):
        pltpu.sync_copy(stage, acc.at[pl.ds(s * RPT + i * RB, RB)])
    pltpu.sync_copy(ones_hbm, ones_v)
    pltpu.sync_copy(edges_hbm.at[1, wid], idx_v)
    plsc.subcore_barrier()

    def body(j, _):
        pltpu.sync_copy(ones_v, acc.at[idx_v.at[j]], add=True)
        return 0

    lax.fori_loop(0, C, body, 0)
    plsc.subcore_barrier()
    for i in range(RPT // RB):
        sl = pl.ds(s * RPT + i * RB, RB)
        pltpu.sync_copy(acc.at[sl], stage)

        @pl.when(c == 0)
        def _():
            pltpu.sync_copy(stage, outA.at[sl])

        @pl.when(c == 1)
        def _():
            pltpu.sync_copy(stage, outB.at[sl])


@functools.partial(
    pl.kernel,
    out_type=(jax.ShapeDtypeStruct((NP, D), jnp.float32),
              jax.ShapeDtypeStruct((NP, D), jnp.float32)),
    mesh=_sc_mesh,
    compiler_params=pltpu.CompilerParams(use_tc_tiling_on_sc=False),
    scratch_types=[
        pltpu.VMEM_SHARED((NP, D), jnp.float32),
        pltpu.VMEM((C, K), jnp.int32),
        pltpu.VMEM((C, K), jnp.int32),
    ]
    + [pltpu.VMEM((K, D), jnp.float32) for _ in range(NBUF)]
    + [pltpu.SemaphoreType.DMA for _ in range(2 * NBUF)],
)
def _sc_scatter(g_hbm, edges_hbm, zeros_hbm, outA, outB,
                acc, src_v, dst_v, *bufs_and_sems):
    rows = bufs_and_sems[:NBUF]
    gsem = bufs_and_sems[NBUF:2 * NBUF]
    ssem = bufs_and_sems[2 * NBUF:]
    c = lax.axis_index("c")
    s = lax.axis_index("s")
    wid = s * NC + c
    pltpu.sync_copy(edges_hbm.at[0, wid], src_v)
    pltpu.sync_copy(edges_hbm.at[1, wid], dst_v)
    pltpu.sync_copy(zeros_hbm, rows[0])
    for i in range(RPT // K):
        pltpu.sync_copy(rows[0], acc.at[pl.ds(s * RPT + i * K, K)])
    for t in range(NBUF):
        pltpu.async_copy(g_hbm.at[src_v.at[t]], rows[t], gsem[t])
    plsc.subcore_barrier()

    def body(i, _):
        ds = []
        for t in range(NBUF):
            j = NBUF * i + t
            pltpu.make_async_copy(g_hbm.at[src_v.at[j]], rows[t], gsem[t]).wait()
            ds.append(pltpu.async_copy(rows[t], acc.at[dst_v.at[j]], ssem[t],
                                       add=True))
        for t in range(NBUF):
            ds[t].wait()

            @pl.when(i < C // NBUF - 1)
            def _():
                j2 = NBUF * i + NBUF + t
                pltpu.async_copy(g_hbm.at[src_v.at[j2]], rows[t], gsem[t])
        return 0

    lax.fori_loop(0, C // NBUF, body, 0)
    plsc.subcore_barrier()
    for i in range(RPT // K):
        sl = pl.ds(s * RPT + i * K, K)
        pltpu.sync_copy(acc.at[sl], rows[0])

        @pl.when(c == 0)
        def _():
            pltpu.sync_copy(rows[0], outA.at[sl])

        @pl.when(c == 1)
        def _():
            pltpu.sync_copy(rows[0], outB.at[sl])


R = 2000
GRID = N // R


def _dis(degA, degB):
    return lax.rsqrt(degA[:, :1] + degB[:, :1] + 1.0)


def _graph_norm(x, sums, w, b, ms, eps=1e-5):
    mean = sums[0:1, :] * (1.0 / N)
    ex2 = sums[1:2, :] * (1.0 / N)
    var = ex2 - mean * mean * ms * (2.0 - ms)
    return w * (x - mean * ms) / jnp.sqrt(var + eps) + b


def _tc_pre_body(x_ref, w_ref, degA_ref, degB_ref, g_ref):
    g = jnp.dot(x_ref[...], w_ref[...], preferred_element_type=jnp.float32)
    g_ref[...] = g * _dis(degA_ref[...], degB_ref[...])


def _conv_fin_phase0(accA_ref, accB_ref, g_ref, degA_ref, degB_ref, b_ref,
                     scr, sums, i, dis):
    o = (accA_ref[...] + accB_ref[...] + g_ref[...]) * dis + b_ref[...]
    scr[pl.ds(i * R, R), :] = o

    @pl.when(i == 0)
    def _():
        sums[...] = jnp.zeros_like(sums)

    sums[0:1, :] += jnp.sum(o, axis=0, keepdims=True)
    sums[1:2, :] += jnp.sum(o * o, axis=0, keepdims=True)


def _conv_fin_body(accA_ref, accB_ref, g_ref, degA_ref, degB_ref, b_ref,
                   w2_ref, gnw_ref, gnb_ref, gnms_ref, out_ref, scr, sums):
    p, i = pl.program_id(0), pl.program_id(1)
    dis = _dis(degA_ref[...], degB_ref[...])

    @pl.when(p == 0)
    def _():
        _conv_fin_phase0(accA_ref, accB_ref, g_ref, degA_ref, degB_ref, b_ref,
                         scr, sums, i, dis)

    @pl.when(p == 1)
    def _():
        y = jnp.maximum(
            _graph_norm(scr[pl.ds(i * R, R), :], sums[...], gnw_ref[...],
                        gnb_ref[...], gnms_ref[...]), 0.0)
        out_ref[...] = jnp.dot(y, w2_ref[...],
                               preferred_element_type=jnp.float32) * dis


def _conv_mlp_body(accA_ref, accB_ref, g_ref, degA_ref, degB_ref, b_ref,
                   gnw_ref, gnb_ref, gnms_ref, lw1_ref, lb1_ref, lw2_ref,
                   lb2_ref, out_ref, scr, sums):
    p, i = pl.program_id(0), pl.program_id(1)
    dis = _dis(degA_ref[...], degB_ref[...])

    @pl.when(p == 0)
    def _():
        _conv_fin_phase0(accA_ref, accB_ref, g_ref, degA_ref, degB_ref, b_ref,
                         scr, sums, i, dis)

    @pl.when(p == 1)
    def _():
        y = jnp.maximum(
            _graph_norm(scr[pl.ds(i * R, R), :], sums[...], gnw_ref[...],
                        gnb_ref[...], gnms_ref[...]), 0.0)
        r = jnp.maximum(
            jnp.dot(y, lw1_ref[...], preferred_element_type=jnp.float32)
            + lb1_ref[...], 0.0)
        out_ref[...] = (jnp.dot(r, lw2_ref[...],
                                preferred_element_type=jnp.float32)
                        + lb2_ref[...])


def _rows1(shape):
    return pl.BlockSpec(shape, lambda i: (i, 0))


def _full1(shape):
    return pl.BlockSpec(shape, lambda i: (0, 0))


def _rows2(shape):
    return pl.BlockSpec(shape, lambda p, i: (i, 0))


def _rows2_p0(shape):
    return pl.BlockSpec(shape, lambda p, i: (i * (1 - p), 0))


def _full2(shape):
    return pl.BlockSpec(shape, lambda p, i: (0, 0))


_tc_pre = pl.pallas_call(
    _tc_pre_body,
    grid=(GRID,),
    in_specs=[_rows1((R, D)), _full1((D, D)), _rows1((R, DEGW)),
              _rows1((R, DEGW))],
    out_specs=_rows1((R, D)),
    out_shape=jax.ShapeDtypeStruct((N, D), jnp.float32),
)

_tc_conv_fin = pl.pallas_call(
    _conv_fin_body,
    grid=(2, GRID),
    in_specs=[_rows2_p0((R, D)), _rows2_p0((R, D)), _rows2_p0((R, D)),
              _rows2((R, DEGW)), _rows2((R, DEGW)), _full2((1, D)),
              _full2((D, D)), _full2((1, D)), _full2((1, D)), _full2((1, D))],
    out_specs=_rows2((R, D)),
    out_shape=jax.ShapeDtypeStruct((N, D), jnp.float32),
    scratch_shapes=[pltpu.VMEM((N, D), jnp.float32),
                    pltpu.VMEM((8, D), jnp.float32)],
)

_tc_conv_mlp = pl.pallas_call(
    _conv_mlp_body,
    grid=(2, GRID),
    in_specs=[_rows2_p0((R, D)), _rows2_p0((R, D)), _rows2_p0((R, D)),
              _rows2((R, DEGW)), _rows2((R, DEGW)), _full2((1, D)),
              _full2((1, D)), _full2((1, D)), _full2((1, D)),
              _full2((D, D)), _full2((1, D)), _full2((D, D)), _full2((1, D))],
    out_specs=_rows2((R, D)),
    out_shape=jax.ShapeDtypeStruct((N, D), jnp.float32),
    scratch_shapes=[pltpu.VMEM((N, D), jnp.float32),
                    pltpu.VMEM((8, D), jnp.float32)],
)


def kernel(x, edge_index, W1, b1, gn1_w, gn1_b, gn1_ms, W2, b2, gn2_w, gn2_b,
           gn2_ms, lW1, lb1, lW2, lb2):
    edges = edge_index.reshape(2, NW, C, K)
    ones16 = jnp.ones((K, DEGW), jnp.float32)
    zeros16 = jnp.zeros((RB, DEGW), jnp.float32)
    zerosD = jnp.zeros((K, D), jnp.float32)
    row = lambda v: v.reshape(1, D)

    degA, degB = _sc_degree(edges, ones16, zeros16)

    g1 = _tc_pre(x, W1, degA, degB)
    a1, p1 = _sc_scatter(g1, edges, zerosD)
    g2 = _tc_conv_fin(a1, p1, g1, degA, degB, row(b1), W2,
                      row(gn1_w), row(gn1_b), row(gn1_ms))
    a2, p2 = _sc_scatter(g2, edges, zerosD)
    return _tc_conv_mlp(a2, p2, g2, degA, degB, row(b2),
                        row(gn2_w), row(gn2_b), row(gn2_ms),
                        lW1, row(lb1), lW2, row(lb2))


# speedup vs baseline: 1.4973x; 1.0011x over previous
import functools

import jax
import jax.numpy as jnp
from jax import lax
from jax.experimental import pallas as pl
from jax.experimental.pallas import tpu as pltpu
from jax.experimental.pallas import tpu_sc as plsc

N = 10000
E = 320000
D = 128

NC = 2
NS = 16
NW = NC * NS
EPW = E // NW
K = 40
NBUF = 5
C = EPW // K
NP = 10240
RPT = NP // NS
RB = 128
DEGW = 8

_sc_mesh = plsc.VectorSubcoreMesh(core_axis_name="c", subcore_axis_name="s")


@functools.partial(
    pl.kernel,
    out_type=(jax.ShapeDtypeStruct((NP, DEGW), jnp.float32),
              jax.ShapeDtypeStruct((NP, DEGW), jnp.float32)),
    mesh=_sc_mesh,
    compiler_params=pltpu.CompilerParams(use_tc_tiling_on_sc=False),
    scratch_types=[
        pltpu.VMEM_SHARED((NP, DEGW), jnp.float32),
        pltpu.VMEM((C, K), jnp.int32),
        pltpu.VMEM((K, DEGW), jnp.float32),
        pltpu.VMEM((RB, DEGW), jnp.float32),
    ],
)
def _sc_degree(edges_hbm, ones_hbm, zeros_hbm, outA, outB, acc, idx_v, ones_v,
               stage):
    c = lax.axis_index("c")
    s = lax.axis_index("s")
    wid = s * NC + c
    pltpu.sync_copy(zeros_hbm, stage)
    for i in range(RPT // RB):
        pltpu.sync_copy(stage, acc.at[pl.ds(s * RPT + i * RB, RB)])
    pltpu.sync_copy(ones_hbm, ones_v)
    pltpu.sync_copy(edges_hbm.at[1, wid], idx_v)
    plsc.subcore_barrier()

    def body(j, _):
        pltpu.sync_copy(ones_v, acc.at[idx_v.at[j]], add=True)
        return 0

    lax.fori_loop(0, C, body, 0)
    plsc.subcore_barrier()
    for i in range(RPT // RB):
        sl = pl.ds(s * RPT + i * RB, RB)
        pltpu.sync_copy(acc.at[sl], stage)

        @pl.when(c == 0)
        def _():
            pltpu.sync_copy(stage, outA.at[sl])

        @pl.when(c == 1)
        def _():
            pltpu.sync_copy(stage, outB.at[sl])


@functools.partial(
    pl.kernel,
    out_type=(jax.ShapeDtypeStruct((NP, D), jnp.float32),
              jax.ShapeDtypeStruct((NP, D), jnp.float32)),
    mesh=_sc_mesh,
    compiler_params=pltpu.CompilerParams(use_tc_tiling_on_sc=False),
    scratch_types=[
        pltpu.VMEM_SHARED((NP, D), jnp.float32),
        pltpu.VMEM((C, K), jnp.int32),
        pltpu.VMEM((C, K), jnp.int32),
    ]
    + [pltpu.VMEM((K, D), jnp.float32) for _ in range(NBUF)]
    + [pltpu.SemaphoreType.DMA for _ in range(2 * NBUF)],
)
def _sc_scatter(g_hbm, edges_hbm, zeros_hbm, outA, outB,
                acc, src_v, dst_v, *bufs_and_sems):
    rows = bufs_and_sems[:NBUF]
    gsem = bufs_and_sems[NBUF:2 * NBUF]
    ssem = bufs_and_sems[2 * NBUF:]
    c = lax.axis_index("c")
    s = lax.axis_index("s")
    wid = s * NC + c
    pltpu.sync_copy(edges_hbm.at[0, wid], src_v)
    pltpu.sync_copy(edges_hbm.at[1, wid], dst_v)
    pltpu.sync_copy(zeros_hbm, rows[0])
    for i in range(RPT // K):
        pltpu.sync_copy(rows[0], acc.at[pl.ds(s * RPT + i * K, K)])
    for t in range(NBUF):
        pltpu.async_copy(g_hbm.at[src_v.at[t]], rows[t], gsem[t])
    plsc.subcore_barrier()

    def body(i, _):
        ds = []
        for t in range(NBUF):
            j = NBUF * i + t
            pltpu.make_async_copy(g_hbm.at[src_v.at[j]], rows[t], gsem[t]).wait()
            ds.append(pltpu.async_copy(rows[t], acc.at[dst_v.at[j]], ssem[t],
                                       add=True))
        for t in range(NBUF):
            ds[t].wait()

            @pl.when(i < C // NBUF - 1)
            def _():
                j2 = NBUF * i + NBUF + t
                pltpu.async_copy(g_hbm.at[src_v.at[j2]], rows[t], gsem[t])
        return 0

    lax.fori_loop(0, C // NBUF, body, 0)
    plsc.subcore_barrier()
    for i in range(RPT // K):
        sl = pl.ds(s * RPT + i * K, K)
        pltpu.sync_copy(acc.at[sl], rows[0])

        @pl.when(c == 0)
        def _():
            pltpu.sync_copy(rows[0], outA.at[sl])

        @pl.when(c == 1)
        def _():
            pltpu.sync_copy(rows[0], outB.at[sl])


R = 2000
GRID = N // R


def _dis(degA, degB):
    return lax.rsqrt(degA[:, :1] + degB[:, :1] + 1.0)


def _graph_norm(x, sums, w, b, ms, eps=1e-5):
    mean = sums[0:1, :] * (1.0 / N)
    ex2 = sums[1:2, :] * (1.0 / N)
    var = ex2 - mean * mean * ms * (2.0 - ms)
    return w * (x - mean * ms) / jnp.sqrt(var + eps) + b


def _tc_mm_body(x_ref, w_ref, h_ref):
    h_ref[...] = jnp.dot(x_ref[...], w_ref[...],
                         preferred_element_type=jnp.float32)


def _tc_scale_body(h_ref, degA_ref, degB_ref, g_ref):
    g_ref[...] = h_ref[...] * _dis(degA_ref[...], degB_ref[...])


def _conv_fin_phase0(accA_ref, accB_ref, g_ref, degA_ref, degB_ref, b_ref,
                     scr, sums, i, dis):
    o = (accA_ref[...] + accB_ref[...] + g_ref[...]) * dis + b_ref[...]
    scr[pl.ds(i * R, R), :] = o

    @pl.when(i == 0)
    def _():
        sums[...] = jnp.zeros_like(sums)

    sums[0:1, :] += jnp.sum(o, axis=0, keepdims=True)
    sums[1:2, :] += jnp.sum(o * o, axis=0, keepdims=True)


def _conv_fin_body(accA_ref, accB_ref, g_ref, degA_ref, degB_ref, b_ref,
                   w2_ref, gnw_ref, gnb_ref, gnms_ref, out_ref, scr, sums):
    p, i = pl.program_id(0), pl.program_id(1)
    dis = _dis(degA_ref[...], degB_ref[...])

    @pl.when(p == 0)
    def _():
        _conv_fin_phase0(accA_ref, accB_ref, g_ref, degA_ref, degB_ref, b_ref,
                         scr, sums, i, dis)

    @pl.when(p == 1)
    def _():
        y = jnp.maximum(
            _graph_norm(scr[pl.ds(i * R, R), :], sums[...], gnw_ref[...],
                        gnb_ref[...], gnms_ref[...]), 0.0)
        out_ref[...] = jnp.dot(y, w2_ref[...],
                               preferred_element_type=jnp.float32) * dis


def _conv_mlp_body(accA_ref, accB_ref, g_ref, degA_ref, degB_ref, b_ref,
                   gnw_ref, gnb_ref, gnms_ref, lw1_ref, lb1_ref, lw2_ref,
                   lb2_ref, out_ref, scr, sums):
    p, i = pl.program_id(0), pl.program_id(1)
    dis = _dis(degA_ref[...], degB_ref[...])

    @pl.when(p == 0)
    def _():
        _conv_fin_phase0(accA_ref, accB_ref, g_ref, degA_ref, degB_ref, b_ref,
                         scr, sums, i, dis)

    @pl.when(p == 1)
    def _():
        y = jnp.maximum(
            _graph_norm(scr[pl.ds(i * R, R), :], sums[...], gnw_ref[...],
                        gnb_ref[...], gnms_ref[...]), 0.0)
        r = jnp.maximum(
            jnp.dot(y, lw1_ref[...], preferred_element_type=jnp.float32)
            + lb1_ref[...], 0.0)
        out_ref[...] = (jnp.dot(r, lw2_ref[...],
                                preferred_element_type=jnp.float32)
                        + lb2_ref[...])


def _rows1(shape):
    return pl.BlockSpec(shape, lambda i: (i, 0))


def _full1(shape):
    return pl.BlockSpec(shape, lambda i: (0, 0))


def _rows2(shape):
    return pl.BlockSpec(shape, lambda p, i: (i, 0))


def _rows2_p0(shape):
    return pl.BlockSpec(shape, lambda p, i: (i * (1 - p), 0))


def _full2(shape):
    return pl.BlockSpec(shape, lambda p, i: (0, 0))


_tc_mm = pl.pallas_call(
    _tc_mm_body,
    grid=(GRID,),
    in_specs=[_rows1((R, D)), _full1((D, D))],
    out_specs=_rows1((R, D)),
    out_shape=jax.ShapeDtypeStruct((N, D), jnp.float32),
)

_tc_scale = pl.pallas_call(
    _tc_scale_body,
    grid=(GRID,),
    in_specs=[_rows1((R, D)), _rows1((R, DEGW)), _rows1((R, DEGW))],
    out_specs=_rows1((R, D)),
    out_shape=jax.ShapeDtypeStruct((N, D), jnp.float32),
)

_tc_conv_fin = pl.pallas_call(
    _conv_fin_body,
    grid=(2, GRID),
    in_specs=[_rows2_p0((R, D)), _rows2_p0((R, D)), _rows2_p0((R, D)),
              _rows2((R, DEGW)), _rows2((R, DEGW)), _full2((1, D)),
              _full2((D, D)), _full2((1, D)), _full2((1, D)), _full2((1, D))],
    out_specs=_rows2((R, D)),
    out_shape=jax.ShapeDtypeStruct((N, D), jnp.float32),
    scratch_shapes=[pltpu.VMEM((N, D), jnp.float32),
                    pltpu.VMEM((8, D), jnp.float32)],
)

_tc_conv_mlp = pl.pallas_call(
    _conv_mlp_body,
    grid=(2, GRID),
    in_specs=[_rows2_p0((R, D)), _rows2_p0((R, D)), _rows2_p0((R, D)),
              _rows2((R, DEGW)), _rows2((R, DEGW)), _full2((1, D)),
              _full2((1, D)), _full2((1, D)), _full2((1, D)),
              _full2((D, D)), _full2((1, D)), _full2((D, D)), _full2((1, D))],
    out_specs=_rows2((R, D)),
    out_shape=jax.ShapeDtypeStruct((N, D), jnp.float32),
    scratch_shapes=[pltpu.VMEM((N, D), jnp.float32),
                    pltpu.VMEM((8, D), jnp.float32)],
)


def kernel(x, edge_index, W1, b1, gn1_w, gn1_b, gn1_ms, W2, b2, gn2_w, gn2_b,
           gn2_ms, lW1, lb1, lW2, lb2):
    edges = edge_index.reshape(2, NW, C, K)
    ones16 = jnp.ones((K, DEGW), jnp.float32)
    zeros16 = jnp.zeros((RB, DEGW), jnp.float32)
    zerosD = jnp.zeros((K, D), jnp.float32)
    row = lambda v: v.reshape(1, D)

    degA, degB = _sc_degree(edges, ones16, zeros16)

    h1 = _tc_mm(x, W1)
    g1 = _tc_scale(h1, degA, degB)
    a1, p1 = _sc_scatter(g1, edges, zerosD)
    g2 = _tc_conv_fin(a1, p1, g1, degA, degB, row(b1), W2,
                      row(gn1_w), row(gn1_b), row(gn1_ms))
    a2, p2 = _sc_scatter(g2, edges, zerosD)
    return _tc_conv_mlp(a2, p2, g2, degA, degB, row(b2),
                        row(gn2_w), row(gn2_b), row(gn2_ms),
                        lW1, row(lb1), lW2, row(lb2))


# speedup vs baseline: 1.5524x; 1.0368x over previous
import functools

import jax
import jax.numpy as jnp
from jax import lax
from jax.experimental import pallas as pl
from jax.experimental.pallas import tpu as pltpu
from jax.experimental.pallas import tpu_sc as plsc

N = 10000
E = 320000
D = 128

NC = 2
NS = 16
NW = NC * NS
EPW = E // NW
K = 40
NBUF = 5
C = EPW // K
NP = 10240
RPT = NP // NS
RB = 128
DEGW = 8

_sc_mesh = plsc.VectorSubcoreMesh(core_axis_name="c", subcore_axis_name="s")


@functools.partial(
    pl.kernel,
    out_type=(jax.ShapeDtypeStruct((NP, DEGW), jnp.float32),
              jax.ShapeDtypeStruct((NP, DEGW), jnp.float32)),
    mesh=_sc_mesh,
    compiler_params=pltpu.CompilerParams(use_tc_tiling_on_sc=False),
    scratch_types=[
        pltpu.VMEM_SHARED((NP, DEGW), jnp.float32),
        pltpu.VMEM((C, K), jnp.int32),
        pltpu.VMEM((K, DEGW), jnp.float32),
        pltpu.VMEM((RB, DEGW), jnp.float32),
    ],
)
def _sc_degree(edges_hbm, ones_hbm, zeros_hbm, outA, outB, acc, idx_v, ones_v,
               stage):
    c = lax.axis_index("c")
    s = lax.axis_index("s")
    wid = s * NC + c
    pltpu.sync_copy(zeros_hbm, stage)
    for i in range(RPT // RB):
        pltpu.sync_copy(stage, acc.at[pl.ds(s * RPT + i * RB, RB)])
    pltpu.sync_copy(ones_hbm, ones_v)
    pltpu.sync_copy(edges_hbm.at[1, wid], idx_v)
    plsc.subcore_barrier()

    def body(j, _):
        pltpu.sync_copy(ones_v, acc.at[idx_v.at[j]], add=True)
        return 0

    lax.fori_loop(0, C, body, 0)
    plsc.subcore_barrier()
    for i in range(RPT // RB):
        sl = pl.ds(s * RPT + i * RB, RB)
        pltpu.sync_copy(acc.at[sl], stage)

        @pl.when(c == 0)
        def _():
            pltpu.sync_copy(stage, outA.at[sl])

        @pl.when(c == 1)
        def _():
            pltpu.sync_copy(stage, outB.at[sl])


@functools.partial(
    pl.kernel,
    out_type=(jax.ShapeDtypeStruct((NP, D), jnp.float32),
              jax.ShapeDtypeStruct((NP, D), jnp.float32)),
    mesh=_sc_mesh,
    compiler_params=pltpu.CompilerParams(use_tc_tiling_on_sc=False),
    scratch_types=[
        pltpu.VMEM_SHARED((NP, D), jnp.float32),
        pltpu.VMEM((C, K), jnp.int32),
        pltpu.VMEM((C, K), jnp.int32),
    ]
    + [pltpu.VMEM((K, D), jnp.float32) for _ in range(NBUF)]
    + [pltpu.SemaphoreType.DMA for _ in range(2 * NBUF)],
)
def _sc_scatter(g_hbm, edges_hbm, zeros_hbm, outA, outB,
                acc, src_v, dst_v, *bufs_and_sems):
    rows = bufs_and_sems[:NBUF]
    gsem = bufs_and_sems[NBUF:2 * NBUF]
    ssem = bufs_and_sems[2 * NBUF:]
    c = lax.axis_index("c")
    s = lax.axis_index("s")
    wid = s * NC + c
    d_src = pltpu.async_copy(edges_hbm.at[0, wid], src_v, ssem[0])
    d_dst = pltpu.async_copy(edges_hbm.at[1, wid], dst_v, ssem[1])
    pltpu.sync_copy(zeros_hbm, rows[0])
    zd = [pltpu.async_copy(rows[0], acc.at[pl.ds(s * RPT + i * K, K)], gsem[0])
          for i in range(RPT // K)]
    d_src.wait()
    d_dst.wait()
    for d in zd:
        d.wait()
    for t in range(NBUF):
        pltpu.async_copy(g_hbm.at[src_v.at[t]], rows[t], gsem[t])
    plsc.subcore_barrier()

    def body(i, _):
        ds = []
        for t in range(NBUF):
            j = NBUF * i + t
            pltpu.make_async_copy(g_hbm.at[src_v.at[j]], rows[t], gsem[t]).wait()
            ds.append(pltpu.async_copy(rows[t], acc.at[dst_v.at[j]], ssem[t],
                                       add=True))
        for t in range(NBUF):
            ds[t].wait()

            @pl.when(i < C // NBUF - 1)
            def _():
                j2 = NBUF * i + NBUF + t
                pltpu.async_copy(g_hbm.at[src_v.at[j2]], rows[t], gsem[t])
        return 0

    lax.fori_loop(0, C // NBUF, body, 0)
    plsc.subcore_barrier()

    WB = RPT // K

    def _slc(i):
        return pl.ds(s * RPT + i * K, K)

    def _fire_out(j):
        t = j % NBUF
        pltpu.make_async_copy(acc.at[_slc(j)], rows[t], gsem[t]).wait()

        @pl.when(c == 0)
        def _():
            pltpu.async_copy(rows[t], outA.at[_slc(j)], ssem[t])

        @pl.when(c == 1)
        def _():
            pltpu.async_copy(rows[t], outB.at[_slc(j)], ssem[t])

    def _wait_out(j):
        t = j % NBUF

        @pl.when(c == 0)
        def _():
            pltpu.make_async_copy(rows[t], outA.at[_slc(j)], ssem[t]).wait()

        @pl.when(c == 1)
        def _():
            pltpu.make_async_copy(rows[t], outB.at[_slc(j)], ssem[t]).wait()

    for i in range(WB):
        if i >= NBUF:
            _wait_out(i - NBUF)
        pltpu.async_copy(acc.at[_slc(i)], rows[i % NBUF], gsem[i % NBUF])
        if i >= 1:
            _fire_out(i - 1)
    _fire_out(WB - 1)
    for j in range(WB - NBUF, WB):
        _wait_out(j)


R = 2000
GRID = N // R


def _graph_norm(x, sums, w, b, ms, eps=1e-5):
    mean = sums[0:1, :] * (1.0 / N)
    ex2 = sums[1:2, :] * (1.0 / N)
    var = ex2 - mean * mean * ms * (2.0 - ms)
    return w * (x - mean * ms) / jnp.sqrt(var + eps) + b


def _tc_mm_body(x_ref, w_ref, h_ref):
    h_ref[...] = jnp.dot(x_ref[...], w_ref[...],
                         preferred_element_type=jnp.float32)


def _tc_scale_body(h_ref, dis_ref, g_ref):
    g_ref[...] = h_ref[...] * dis_ref[:, :1]


def _conv_fin_phase0(accA_ref, accB_ref, g_ref, b_ref, scr, sums, i, dis):
    o = (accA_ref[...] + accB_ref[...] + g_ref[...]) * dis + b_ref[...]
    scr[pl.ds(i * R, R), :] = o

    @pl.when(i == 0)
    def _():
        sums[...] = jnp.zeros_like(sums)

    sums[0:1, :] += jnp.sum(o, axis=0, keepdims=True)
    sums[1:2, :] += jnp.sum(o * o, axis=0, keepdims=True)


def _conv_fin_body(accA_ref, accB_ref, g_ref, dis_ref, b_ref,
                   w2_ref, gnw_ref, gnb_ref, gnms_ref, out_ref, scr, sums):
    p, i = pl.program_id(0), pl.program_id(1)
    dis = dis_ref[:, :1]

    @pl.when(p == 0)
    def _():
        _conv_fin_phase0(accA_ref, accB_ref, g_ref, b_ref, scr, sums, i, dis)

    @pl.when(p == 1)
    def _():
        y = jnp.maximum(
            _graph_norm(scr[pl.ds(i * R, R), :], sums[...], gnw_ref[...],
                        gnb_ref[...], gnms_ref[...]), 0.0)
        out_ref[...] = jnp.dot(y, w2_ref[...],
                               preferred_element_type=jnp.float32) * dis


def _conv_mlp_body(accA_ref, accB_ref, g_ref, dis_ref, b_ref,
                   gnw_ref, gnb_ref, gnms_ref, lw1_ref, lb1_ref, lw2_ref,
                   lb2_ref, out_ref, scr, sums):
    p, i = pl.program_id(0), pl.program_id(1)
    dis = dis_ref[:, :1]

    @pl.when(p == 0)
    def _():
        _conv_fin_phase0(accA_ref, accB_ref, g_ref, b_ref, scr, sums, i, dis)

    @pl.when(p == 1)
    def _():
        y = jnp.maximum(
            _graph_norm(scr[pl.ds(i * R, R), :], sums[...], gnw_ref[...],
                        gnb_ref[...], gnms_ref[...]), 0.0)
        r = jnp.maximum(
            jnp.dot(y, lw1_ref[...], preferred_element_type=jnp.float32)
            + lb1_ref[...], 0.0)
        out_ref[...] = (jnp.dot(r, lw2_ref[...],
                                preferred_element_type=jnp.float32)
                        + lb2_ref[...])


def _rows1(shape):
    return pl.BlockSpec(shape, lambda i: (i, 0))


def _full1(shape):
    return pl.BlockSpec(shape, lambda i: (0, 0))


def _rows2(shape):
    return pl.BlockSpec(shape, lambda p, i: (i, 0))


def _rows2_p0(shape):
    return pl.BlockSpec(shape, lambda p, i: (i * (1 - p), 0))


def _full2(shape):
    return pl.BlockSpec(shape, lambda p, i: (0, 0))


_tc_mm = pl.pallas_call(
    _tc_mm_body,
    grid=(GRID,),
    in_specs=[_rows1((R, D)), _full1((D, D))],
    out_specs=_rows1((R, D)),
    out_shape=jax.ShapeDtypeStruct((N, D), jnp.float32),
)

_tc_scale = pl.pallas_call(
    _tc_scale_body,
    grid=(GRID,),
    in_specs=[_rows1((R, D)), _rows1((R, DEGW))],
    out_specs=_rows1((R, D)),
    out_shape=jax.ShapeDtypeStruct((N, D), jnp.float32),
)

_tc_conv_fin = pl.pallas_call(
    _conv_fin_body,
    grid=(2, GRID),
    in_specs=[_rows2_p0((R, D)), _rows2_p0((R, D)), _rows2_p0((R, D)),
              _rows2((R, DEGW)), _full2((1, D)),
              _full2((D, D)), _full2((1, D)), _full2((1, D)), _full2((1, D))],
    out_specs=_rows2((R, D)),
    out_shape=jax.ShapeDtypeStruct((N, D), jnp.float32),
    scratch_shapes=[pltpu.VMEM((N, D), jnp.float32),
                    pltpu.VMEM((8, D), jnp.float32)],
)

_tc_conv_mlp = pl.pallas_call(
    _conv_mlp_body,
    grid=(2, GRID),
    in_specs=[_rows2_p0((R, D)), _rows2_p0((R, D)), _rows2_p0((R, D)),
              _rows2((R, DEGW)), _full2((1, D)),
              _full2((1, D)), _full2((1, D)), _full2((1, D)),
              _full2((D, D)), _full2((1, D)), _full2((D, D)), _full2((1, D))],
    out_specs=_rows2((R, D)),
    out_shape=jax.ShapeDtypeStruct((N, D), jnp.float32),
    scratch_shapes=[pltpu.VMEM((N, D), jnp.float32),
                    pltpu.VMEM((8, D), jnp.float32)],
)


def kernel(x, edge_index, W1, b1, gn1_w, gn1_b, gn1_ms, W2, b2, gn2_w, gn2_b,
           gn2_ms, lW1, lb1, lW2, lb2):
    edges = edge_index.reshape(2, NW, C, K)
    ones16 = jnp.ones((K, DEGW), jnp.float32)
    zeros16 = jnp.zeros((RB, DEGW), jnp.float32)
    zerosD = jnp.zeros((K, D), jnp.float32)
    row = lambda v: v.reshape(1, D)

    degA, degB = _sc_degree(edges, ones16, zeros16)
    dis8 = lax.rsqrt(degA + degB + 1.0)

    h1 = _tc_mm(x, W1)
    g1 = _tc_scale(h1, dis8)
    a1, p1 = _sc_scatter(g1, edges, zerosD)
    g2 = _tc_conv_fin(a1, p1, g1, dis8, row(b1), W2,
                      row(gn1_w), row(gn1_b), row(gn1_ms))
    a2, p2 = _sc_scatter(g2, edges, zerosD)
    return _tc_conv_mlp(a2, p2, g2, dis8, row(b2),
                        row(gn2_w), row(gn2_b), row(gn2_ms),
                        lW1, row(lb1), lW2, row(lb2))


# speedup vs baseline: 1.6386x; 1.0556x over previous
import functools

import jax
import jax.numpy as jnp
from jax import lax
from jax.experimental import pallas as pl
from jax.experimental.pallas import tpu as pltpu
from jax.experimental.pallas import tpu_sc as plsc

N = 10000
E = 320000
D = 128

NC = 2
NS = 16
NW = NC * NS
EPW = E // NW
K = 40
NBUF = 5
C = EPW // K
NP = 10240
RPT = NP // NS
RB = 128
DEGW = 8

_sc_mesh = plsc.VectorSubcoreMesh(core_axis_name="c", subcore_axis_name="s")


KD = K
CD = C
NDS = 5


@functools.partial(
    pl.kernel,
    out_type=(jax.ShapeDtypeStruct((NP, DEGW), jnp.float32),
              jax.ShapeDtypeStruct((NP, DEGW), jnp.float32)),
    mesh=_sc_mesh,
    compiler_params=pltpu.CompilerParams(use_tc_tiling_on_sc=False),
    scratch_types=[
        pltpu.VMEM_SHARED((NP, DEGW), jnp.float32),
        pltpu.VMEM((CD, KD), jnp.int32),
        pltpu.VMEM((KD, DEGW), jnp.float32),
        pltpu.VMEM((RB, DEGW), jnp.float32),
        pltpu.VMEM((RB, DEGW), jnp.float32),
    ]
    + [pltpu.SemaphoreType.DMA for _ in range(NDS + 2)],
)
def _sc_degree(edges_hbm, ones_hbm, zeros_hbm, outA, outB, acc, idx_v, ones_v,
               stg0, stg1, *sems):
    dsem = sems[:NDS]
    osem = sems[NDS:]
    stg = (stg0, stg1)
    c = lax.axis_index("c")
    s = lax.axis_index("s")
    wid = s * NC + c
    d_idx = pltpu.async_copy(edges_hbm.at[1, wid], idx_v, dsem[1])
    d_one = pltpu.async_copy(ones_hbm, ones_v, dsem[2])
    pltpu.sync_copy(zeros_hbm, stg0)
    zd = [pltpu.async_copy(stg0, acc.at[pl.ds(s * RPT + i * RB, RB)], dsem[0])
          for i in range(RPT // RB)]
    d_idx.wait()
    d_one.wait()
    for d in zd:
        d.wait()
    plsc.subcore_barrier()

    for t in range(NDS):
        pltpu.async_copy(ones_v, acc.at[idx_v.at[t]], dsem[t], add=True)

    def body(i, _):
        for t in range(NDS):
            jp = NDS * (i - 1) + t
            pltpu.make_async_copy(ones_v, acc.at[idx_v.at[jp]],
                                  dsem[t]).wait()
            pltpu.async_copy(ones_v, acc.at[idx_v.at[NDS * i + t]], dsem[t],
                             add=True)
        return 0

    lax.fori_loop(1, CD // NDS, body, 0)
    for t in range(NDS):
        jp = NDS * (CD // NDS - 1) + t
        pltpu.make_async_copy(ones_v, acc.at[idx_v.at[jp]], dsem[t]).wait()
    plsc.subcore_barrier()

    WBD = RPT // RB

    def _slc(i):
        return pl.ds(s * RPT + i * RB, RB)

    def _fire_out(j):
        t = j % 2
        pltpu.make_async_copy(acc.at[_slc(j)], stg[t], dsem[t]).wait()

        @pl.when(c == 0)
        def _():
            pltpu.async_copy(stg[t], outA.at[_slc(j)], osem[t])

        @pl.when(c == 1)
        def _():
            pltpu.async_copy(stg[t], outB.at[_slc(j)], osem[t])

    def _wait_out(j):
        t = j % 2

        @pl.when(c == 0)
        def _():
            pltpu.make_async_copy(stg[t], outA.at[_slc(j)], osem[t]).wait()

        @pl.when(c == 1)
        def _():
            pltpu.make_async_copy(stg[t], outB.at[_slc(j)], osem[t]).wait()

    for i in range(WBD):
        if i >= 2:
            _wait_out(i - 2)
        pltpu.async_copy(acc.at[_slc(i)], stg[i % 2], dsem[i % 2])
        if i >= 1:
            _fire_out(i - 1)
    _fire_out(WBD - 1)
    for j in range(WBD - 2, WBD):
        _wait_out(j)


@functools.partial(
    pl.kernel,
    out_type=(jax.ShapeDtypeStruct((NP, D), jnp.float32),
              jax.ShapeDtypeStruct((NP, D), jnp.float32)),
    mesh=_sc_mesh,
    compiler_params=pltpu.CompilerParams(use_tc_tiling_on_sc=False),
    scratch_types=[
        pltpu.VMEM_SHARED((NP, D), jnp.float32),
        pltpu.VMEM((C, K), jnp.int32),
        pltpu.VMEM((C, K), jnp.int32),
    ]
    + [pltpu.VMEM((K, D), jnp.float32) for _ in range(NBUF)]
    + [pltpu.SemaphoreType.DMA for _ in range(2 * NBUF)],
)
def _sc_scatter(g_hbm, edges_hbm, zeros_hbm, outA, outB,
                acc, src_v, dst_v, *bufs_and_sems):
    rows = bufs_and_sems[:NBUF]
    gsem = bufs_and_sems[NBUF:2 * NBUF]
    ssem = bufs_and_sems[2 * NBUF:]
    c = lax.axis_index("c")
    s = lax.axis_index("s")
    wid = s * NC + c
    d_src = pltpu.async_copy(edges_hbm.at[0, wid], src_v, ssem[0])
    d_dst = pltpu.async_copy(edges_hbm.at[1, wid], dst_v, ssem[1])
    pltpu.sync_copy(zeros_hbm, rows[0])
    zd = [pltpu.async_copy(rows[0], acc.at[pl.ds(s * RPT + i * K, K)], gsem[0])
          for i in range(RPT // K)]
    d_src.wait()
    d_dst.wait()
    for d in zd:
        d.wait()
    for t in range(NBUF):
        pltpu.async_copy(g_hbm.at[src_v.at[t]], rows[t], gsem[t])
    plsc.subcore_barrier()

    def body(i, _):
        ds = []
        for t in range(NBUF):
            j = NBUF * i + t
            pltpu.make_async_copy(g_hbm.at[src_v.at[j]], rows[t], gsem[t]).wait()
            ds.append(pltpu.async_copy(rows[t], acc.at[dst_v.at[j]], ssem[t],
                                       add=True))
        for t in range(NBUF):
            ds[t].wait()

            @pl.when(i < C // NBUF - 1)
            def _():
                j2 = NBUF * i + NBUF + t
                pltpu.async_copy(g_hbm.at[src_v.at[j2]], rows[t], gsem[t])
        return 0

    lax.fori_loop(0, C // NBUF, body, 0)
    plsc.subcore_barrier()

    WB = RPT // K

    def _slc(i):
        return pl.ds(s * RPT + i * K, K)

    def _fire_out(j):
        t = j % NBUF
        pltpu.make_async_copy(acc.at[_slc(j)], rows[t], gsem[t]).wait()

        @pl.when(c == 0)
        def _():
            pltpu.async_copy(rows[t], outA.at[_slc(j)], ssem[t])

        @pl.when(c == 1)
        def _():
            pltpu.async_copy(rows[t], outB.at[_slc(j)], ssem[t])

    def _wait_out(j):
        t = j % NBUF

        @pl.when(c == 0)
        def _():
            pltpu.make_async_copy(rows[t], outA.at[_slc(j)], ssem[t]).wait()

        @pl.when(c == 1)
        def _():
            pltpu.make_async_copy(rows[t], outB.at[_slc(j)], ssem[t]).wait()

    for i in range(WB):
        if i >= NBUF:
            _wait_out(i - NBUF)
        pltpu.async_copy(acc.at[_slc(i)], rows[i % NBUF], gsem[i % NBUF])
        if i >= 1:
            _fire_out(i - 1)
    _fire_out(WB - 1)
    for j in range(WB - NBUF, WB):
        _wait_out(j)


R = 2000
GRID = N // R


def _graph_norm(x, sums, w, b, ms, eps=1e-5):
    mean = sums[0:1, :] * (1.0 / N)
    ex2 = sums[1:2, :] * (1.0 / N)
    var = ex2 - mean * mean * ms * (2.0 - ms)
    return w * (x - mean * ms) / jnp.sqrt(var + eps) + b


def _tc_mm_body(x_ref, w_ref, h_ref):
    h_ref[...] = jnp.dot(x_ref[...], w_ref[...],
                         preferred_element_type=jnp.float32)


def _tc_scale_body(h_ref, dis_ref, g_ref):
    g_ref[...] = h_ref[...] * dis_ref[:, :1]


def _conv_fin_phase0(accA_ref, accB_ref, g_ref, b_ref, scr, sums, i, dis):
    o = (accA_ref[...] + accB_ref[...] + g_ref[...]) * dis + b_ref[...]
    scr[pl.ds(i * R, R), :] = o

    @pl.when(i == 0)
    def _():
        sums[...] = jnp.zeros_like(sums)

    sums[0:1, :] += jnp.sum(o, axis=0, keepdims=True)
    sums[1:2, :] += jnp.sum(o * o, axis=0, keepdims=True)


def _conv_fin_body(accA_ref, accB_ref, g_ref, dis_ref, b_ref,
                   w2_ref, gnw_ref, gnb_ref, gnms_ref, out_ref, scr, sums):
    p, i = pl.program_id(0), pl.program_id(1)
    dis = dis_ref[:, :1]

    @pl.when(p == 0)
    def _():
        _conv_fin_phase0(accA_ref, accB_ref, g_ref, b_ref, scr, sums, i, dis)

    @pl.when(p == 1)
    def _():
        y = jnp.maximum(
            _graph_norm(scr[pl.ds(i * R, R), :], sums[...], gnw_ref[...],
                        gnb_ref[...], gnms_ref[...]), 0.0)
        out_ref[...] = jnp.dot(y, w2_ref[...],
                               preferred_element_type=jnp.float32) * dis


def _conv_mlp_body(accA_ref, accB_ref, g_ref, dis_ref, b_ref,
                   gnw_ref, gnb_ref, gnms_ref, lw1_ref, lb1_ref, lw2_ref,
                   lb2_ref, out_ref, scr, sums):
    p, i = pl.program_id(0), pl.program_id(1)
    dis = dis_ref[:, :1]

    @pl.when(p == 0)
    def _():
        _conv_fin_phase0(accA_ref, accB_ref, g_ref, b_ref, scr, sums, i, dis)

    @pl.when(p == 1)
    def _():
        y = jnp.maximum(
            _graph_norm(scr[pl.ds(i * R, R), :], sums[...], gnw_ref[...],
                        gnb_ref[...], gnms_ref[...]), 0.0)
        r = jnp.maximum(
            jnp.dot(y, lw1_ref[...], preferred_element_type=jnp.float32)
            + lb1_ref[...], 0.0)
        out_ref[...] = (jnp.dot(r, lw2_ref[...],
                                preferred_element_type=jnp.float32)
                        + lb2_ref[...])


def _rows1(shape):
    return pl.BlockSpec(shape, lambda i: (i, 0))


def _full1(shape):
    return pl.BlockSpec(shape, lambda i: (0, 0))


def _rows2(shape):
    return pl.BlockSpec(shape, lambda p, i: (i, 0))


def _rows2_p0(shape):
    return pl.BlockSpec(shape, lambda p, i: (i * (1 - p), 0))


def _full2(shape):
    return pl.BlockSpec(shape, lambda p, i: (0, 0))


_tc_mm = pl.pallas_call(
    _tc_mm_body,
    grid=(GRID,),
    in_specs=[_rows1((R, D)), _full1((D, D))],
    out_specs=_rows1((R, D)),
    out_shape=jax.ShapeDtypeStruct((N, D), jnp.float32),
)

_tc_scale = pl.pallas_call(
    _tc_scale_body,
    grid=(GRID,),
    in_specs=[_rows1((R, D)), _rows1((R, DEGW))],
    out_specs=_rows1((R, D)),
    out_shape=jax.ShapeDtypeStruct((N, D), jnp.float32),
)

_tc_conv_fin = pl.pallas_call(
    _conv_fin_body,
    grid=(2, GRID),
    in_specs=[_rows2_p0((R, D)), _rows2_p0((R, D)), _rows2_p0((R, D)),
              _rows2((R, DEGW)), _full2((1, D)),
              _full2((D, D)), _full2((1, D)), _full2((1, D)), _full2((1, D))],
    out_specs=_rows2((R, D)),
    out_shape=jax.ShapeDtypeStruct((N, D), jnp.float32),
    scratch_shapes=[pltpu.VMEM((N, D), jnp.float32),
                    pltpu.VMEM((8, D), jnp.float32)],
)

_tc_conv_mlp = pl.pallas_call(
    _conv_mlp_body,
    grid=(2, GRID),
    in_specs=[_rows2_p0((R, D)), _rows2_p0((R, D)), _rows2_p0((R, D)),
              _rows2((R, DEGW)), _full2((1, D)),
              _full2((1, D)), _full2((1, D)), _full2((1, D)),
              _full2((D, D)), _full2((1, D)), _full2((D, D)), _full2((1, D))],
    out_specs=_rows2((R, D)),
    out_shape=jax.ShapeDtypeStruct((N, D), jnp.float32),
    scratch_shapes=[pltpu.VMEM((N, D), jnp.float32),
                    pltpu.VMEM((8, D), jnp.float32)],
)


def kernel(x, edge_index, W1, b1, gn1_w, gn1_b, gn1_ms, W2, b2, gn2_w, gn2_b,
           gn2_ms, lW1, lb1, lW2, lb2):
    edges = edge_index.reshape(2, NW, C, K)
    ones16 = jnp.ones((KD, DEGW), jnp.float32)
    zeros16 = jnp.zeros((RB, DEGW), jnp.float32)
    zerosD = jnp.zeros((K, D), jnp.float32)
    row = lambda v: v.reshape(1, D)

    degA, degB = _sc_degree(edges, ones16, zeros16)
    dis8 = lax.rsqrt(degA + degB + 1.0)

    h1 = _tc_mm(x, W1)
    g1 = _tc_scale(h1, dis8)
    a1, p1 = _sc_scatter(g1, edges, zerosD)
    g2 = _tc_conv_fin(a1, p1, g1, dis8, row(b1), W2,
                      row(gn1_w), row(gn1_b), row(gn1_ms))
    a2, p2 = _sc_scatter(g2, edges, zerosD)
    return _tc_conv_mlp(a2, p2, g2, dis8, row(b2),
                        row(gn2_w), row(gn2_b), row(gn2_ms),
                        lW1, row(lb1), lW2, row(lb2))
